# baseline (speedup 1.0000x reference)
import functools
import math

import jax
import jax.numpy as jnp
from jax import lax
from jax.experimental import pallas as pl
from jax.experimental.pallas import tpu as pltpu
from jax.experimental.pallas import tpu_sc as plsc

D_MODEL = 64
SCALE = math.sqrt(D_MODEL)
PAD_D = 128

_NC = 2
_NS = 16
_L = 16
_NW = _NC * _NS

_NBUF = 4
_LEAD = 2
_GMAX = 128


@functools.lru_cache(maxsize=None)
def _make_format(vocab):
    bw = 256
    n_blocks = vocab // bw
    tail = vocab - n_blocks * bw
    out_rows = vocab
    per_w = n_blocks // _NW
    extra = n_blocks - per_w * _NW
    mesh = plsc.VectorSubcoreMesh(core_axis_name="c", subcore_axis_name="s")

    scratch = [pltpu.VMEM((D_MODEL, bw), jnp.float32) for _ in range(2)]
    scratch += [pltpu.VMEM((bw, PAD_D), jnp.float32) for _ in range(2)]
    if tail:
        scratch += [
            pltpu.VMEM((D_MODEL, tail), jnp.float32),
            pltpu.VMEM((tail, PAD_D), jnp.float32),
        ]
    scratch += [pltpu.SemaphoreType.DMA for _ in range(4)]

    @functools.partial(
        pl.kernel,
        mesh=mesh,
        out_type=jax.ShapeDtypeStruct((out_rows, PAD_D), jnp.float32),
        scratch_types=scratch,
        compiler_params=pltpu.CompilerParams(needs_layout_passes=False),
    )
    def fmt(tab_t, tail_t, out_hbm, *bufs):
        tiles = bufs[0:2]
        obufs = bufs[2:4]
        if tail:
            ttile, otail = bufs[4:6]
            sems = bufs[6:]
        else:
            sems = bufs[4:]
        sem_in = sems[0:2]
        sem_out = sems[2:4]
        wid = lax.axis_index("s") * _NC + lax.axis_index("c")
        n_t = jnp.where(wid < extra, per_w + 1, per_w)
        iota = lax.iota(jnp.int32, _L)
        idx_f = [j * _L + iota for j in range(D_MODEL // _L)]

        def off(t):
            return pl.multiple_of((wid + t * _NW) * bw, bw)

        def fire_in(t, b):
            pltpu.async_copy(
                tab_t.at[:, pl.ds(off(t), bw)], tiles[b], sem_in[b]
            )

        def wait_in(b):
            pltpu.make_async_copy(
                tab_t.at[:, pl.ds(0, bw)], tiles[b], sem_in[b]
            ).wait()

        def fire_out(t, b):
            pltpu.async_copy(
                obufs[b], out_hbm.at[pl.ds(off(t), bw)], sem_out[b]
            )

        def wait_out(b):
            pltpu.make_async_copy(
                obufs[b], out_hbm.at[pl.ds(0, bw)], sem_out[b]
            ).wait()

        def transpose_into(src, dst, width):
            @plsc.parallel_loop(
                0, width, step=1, unroll=8, carry=jnp.zeros((_L,), jnp.int32)
            )
            def _rows(r, idx_c):
                for j in range(D_MODEL // _L):
                    v = plsc.load_gather(src, [idx_f[j], idx_c])
                    dst[r, pl.ds(j * _L, _L)] = v * SCALE
                return idx_c + 1

        fire_in(0, 0)

        def superstep(c, _):
            for b in range(2):
                t = c * 2 + b

                @pl.when(t < n_t)
                def _():
                    @pl.when(t + 1 < n_t)
                    def _():
                        fire_in(t + 1, 1 - b)

                    wait_in(b)

                    @pl.when(t >= 2)
                    def _():
                        wait_out(b)

                    transpose_into(tiles[b], obufs[b], bw)
                    fire_out(t, b)

            return 0

        lax.fori_loop(0, (per_w + 2) // 2, superstep, 0)

        wait_out(0)
        wait_out(1)

        if tail:
            @pl.when(wid == _NW - 1)
            def _tail():
                pltpu.sync_copy(tail_t, ttile)
                transpose_into(ttile, otail, tail)
                pltpu.sync_copy(
                    otail, out_hbm.at[pl.ds(n_blocks * bw, tail)]
                )

    return fmt


@functools.lru_cache(maxsize=None)
def _make_lookup(vocab, n_rows, n_cols):
    rows_per_w = n_rows // _NW
    splits = []
    c0 = 0
    while c0 < n_cols:
        g = min(_GMAX, n_cols - c0)
        splits.append((c0, g))
        c0 += g
    mesh = plsc.VectorSubcoreMesh(core_axis_name="c", subcore_axis_name="s")

    scratch = [pltpu.VMEM((rows_per_w, n_cols), jnp.int32)]
    scratch += [pltpu.VMEM((n_cols, PAD_D), jnp.float32) for _ in range(_NBUF)]
    scratch += [pltpu.SemaphoreType.DMA for _ in range(2 * _NBUF)]

    @functools.partial(
        pl.kernel,
        mesh=mesh,
        out_type=jax.ShapeDtypeStruct((n_rows, n_cols, PAD_D), jnp.float32),
        scratch_types=scratch,
        compiler_params=pltpu.CompilerParams(use_tc_tiling_on_sc=False),
    )
    def lookup(table_hbm, idx_hbm, out_hbm, idx_v, *bufs):
        rows = bufs[:_NBUF]
        sem_in = bufs[_NBUF:2 * _NBUF]
        sem_out = bufs[2 * _NBUF:]
        wid = lax.axis_index("s") * _NC + lax.axis_index("c")
        base = wid * rows_per_w
        pltpu.sync_copy(idx_hbm.at[pl.ds(base, rows_per_w)], idx_v)

        def fire_gather(g, b):
            for (c0, gw) in splits:
                pltpu.async_copy(
                    table_hbm.at[idx_v.at[g, pl.ds(c0, gw)]],
                    rows[b].at[pl.ds(c0, gw)],
                    sem_in[b],
                )

        def wait_gather(b):
            pltpu.make_async_copy(
                table_hbm.at[pl.ds(0, n_cols)], rows[b], sem_in[b]
            ).wait()

        def fire_writeback(g, b):
            pltpu.async_copy(
                rows[b].at[:, pl.ds(0, D_MODEL)],
                out_hbm.at[base + g, :, pl.ds(0, D_MODEL)],
                sem_out[b],
            )

        def wait_writeback(b):
            pltpu.make_async_copy(
                rows[b].at[:, pl.ds(0, D_MODEL)],
                out_hbm.at[0, :, pl.ds(0, D_MODEL)],
                sem_out[b],
            ).wait()

        for g in range(_LEAD):
            fire_gather(g, g % _NBUF)

        def superstep(c, _):
            for b in range(_NBUF):
                g = c * _NBUF + b
                gf = g + _LEAD
                bf = (b + _LEAD) % _NBUF

                @pl.when(gf < rows_per_w)
                def _fire():
                    @pl.when(gf >= _NBUF)
                    def _wb():
                        wait_writeback(bf)

                    fire_gather(gf, bf)

                wait_gather(b)
                fire_writeback(g, b)
            return 0

        lax.fori_loop(0, rows_per_w // _NBUF, superstep, 0)

        for b in range(_NBUF):
            wait_writeback(b)

    return lookup


def kernel(x, embedding):
    n_rows, n_cols = x.shape
    vocab = embedding.shape[0]
    idx = jnp.clip(x.astype(jnp.int32), 0, vocab - 1)
    tab_t = embedding.T
    tail = vocab % PAD_D
    tail_t = tab_t[:, vocab - tail:] if tail else tab_t[:, :8]
    table = _make_format(vocab)(tab_t, tail_t)
    out = _make_lookup(vocab, n_rows, n_cols)(table, idx)
    return out[:, :, :D_MODEL]


# speedup vs baseline: 1.2893x; 1.2893x over previous
import functools
import math

import jax
import jax.numpy as jnp
from jax import lax
from jax.experimental import pallas as pl
from jax.experimental.pallas import tpu as pltpu
from jax.experimental.pallas import tpu_sc as plsc

D_MODEL = 64
SCALE = math.sqrt(D_MODEL)
PAD_D = 128

_NC = 2
_NS = 16
_L = 16
_NW = _NC * _NS

_NBUF = 4
_LEAD = 2
_GMAX = 128


@functools.lru_cache(maxsize=None)
def _make_lookup(vocab, n_rows, n_cols):
    rows_per_w = n_rows // _NW
    splits = []
    c0 = 0
    while c0 < n_cols:
        g = min(_GMAX, n_cols - c0)
        splits.append((c0, g))
        c0 += g
    mesh = plsc.VectorSubcoreMesh(core_axis_name="c", subcore_axis_name="s")

    scratch = [pltpu.VMEM((rows_per_w, n_cols), jnp.int32)]
    scratch += [pltpu.VMEM((n_cols, D_MODEL), jnp.float32) for _ in range(_NBUF)]
    scratch += [pltpu.SemaphoreType.DMA for _ in range(2 * _NBUF)]

    @functools.partial(
        pl.kernel,
        mesh=mesh,
        out_type=jax.ShapeDtypeStruct((n_rows, n_cols, PAD_D), jnp.float32),
        scratch_types=scratch,
        compiler_params=pltpu.CompilerParams(use_tc_tiling_on_sc=False),
    )
    def lookup(table_hbm, idx_hbm, out_hbm, idx_v, *bufs):
        rows = bufs[:_NBUF]
        sem_in = bufs[_NBUF:2 * _NBUF]
        sem_out = bufs[2 * _NBUF:]
        wid = lax.axis_index("s") * _NC + lax.axis_index("c")
        base = wid * rows_per_w
        pltpu.sync_copy(idx_hbm.at[pl.ds(base, rows_per_w)], idx_v)

        def fire_gather(g, b):
            for (c0, gw) in splits:
                pltpu.async_copy(
                    table_hbm.at[idx_v.at[g, pl.ds(c0, gw)]],
                    rows[b].at[pl.ds(c0, gw)],
                    sem_in[b],
                )

        def wait_gather(b):
            pltpu.make_async_copy(
                table_hbm.at[pl.ds(0, n_cols)], rows[b], sem_in[b]
            ).wait()

        def fire_writeback(g, b):
            pltpu.async_copy(
                rows[b], out_hbm.at[base + g, :, pl.ds(0, D_MODEL)], sem_out[b]
            )

        def wait_writeback(b):
            pltpu.make_async_copy(
                rows[b], out_hbm.at[0, :, pl.ds(0, D_MODEL)], sem_out[b]
            ).wait()

        for g in range(_LEAD):
            fire_gather(g, g % _NBUF)

        def superstep(c, _):
            for b in range(_NBUF):
                g = c * _NBUF + b
                gf = g + _LEAD
                bf = (b + _LEAD) % _NBUF

                @pl.when(gf < rows_per_w)
                def _fire():
                    @pl.when(gf >= _NBUF)
                    def _wb():
                        wait_writeback(bf)

                    fire_gather(gf, bf)

                wait_gather(b)

                @plsc.parallel_loop(0, n_cols, step=1, unroll=8)
                def _scale(i):
                    for j in range(D_MODEL // _L):
                        sl = pl.ds(j * _L, _L)
                        rows[b][i, sl] = rows[b][i, sl] * SCALE

                fire_writeback(g, b)
            return 0

        lax.fori_loop(0, rows_per_w // _NBUF, superstep, 0)

        for b in range(_NBUF):
            wait_writeback(b)

    return lookup


def kernel(x, embedding):
    n_rows, n_cols = x.shape
    vocab = embedding.shape[0]
    idx = jnp.clip(x.astype(jnp.int32), 0, vocab - 1)
    out = _make_lookup(vocab, n_rows, n_cols)(embedding, idx)
    return out[:, :, :D_MODEL]
